# baseline (speedup 1.0000x reference)
import functools

import jax
import jax.numpy as jnp
from jax import lax
from jax.experimental import pallas as pl
from jax.experimental.pallas import tpu as pltpu
from jax.experimental.pallas import tpu_sc as plsc

BATCH = 4096
SEQ = 200
EMBED = 128
LANES = 16
NVEC = EMBED // LANES

CHUNK = 128
NB = 4
UNROLL = 8


def _make_kernel(num_cores, num_subcores):
    nw = num_cores * num_subcores
    rows_w = BATCH * SEQ // nw
    b_per_w = BATCH // nw
    t_chunks = rows_w // CHUNK
    groups = t_chunks // NB

    mesh = plsc.VectorSubcoreMesh(core_axis_name="c", subcore_axis_name="s")

    @functools.partial(
        pl.kernel,
        mesh=mesh,
        out_type=jax.ShapeDtypeStruct((BATCH * SEQ, EMBED), jnp.float32),
        scratch_types=[
            pltpu.VMEM((b_per_w,), jnp.int32),
            pltpu.VMEM((b_per_w, EMBED), jnp.float32),
            pltpu.VMEM((SEQ, EMBED), jnp.float32),
        ]
        + [pltpu.VMEM((CHUNK,), jnp.int32) for _ in range(NB)]
        + [pltpu.VMEM((CHUNK, EMBED), jnp.float32) for _ in range(NB)]
        + [pltpu.SemaphoreType.DMA for _ in range(3 * NB + 1)],
    )
    def k(ids_hbm, labels_hbm, item_hbm, ltab_hbm, pos_hbm, out_hbm, *scr):
        labs_v, user_v, pos_v = scr[:3]
        idsb = scr[3:3 + NB]
        buf = scr[3 + NB:3 + 2 * NB]
        sem_i = scr[3 + 2 * NB:3 + 3 * NB]
        sem_g = scr[3 + 3 * NB:3 + 4 * NB]
        sem_w = scr[3 + 4 * NB:3 + 5 * NB]
        sem0 = scr[3 + 5 * NB]

        wid = lax.axis_index("s") * num_cores + lax.axis_index("c")
        r0 = wid * rows_w
        b0 = wid * b_per_w

        pltpu.sync_copy(labels_hbm.at[pl.ds(b0, b_per_w)], labs_v)
        pltpu.sync_copy(pos_hbm, pos_v)
        pltpu.async_copy(ltab_hbm.at[labs_v], user_v, sem0).wait()

        def ids_dma(c, si):
            pltpu.async_copy(ids_hbm.at[pl.ds(r0 + c * CHUNK, CHUNK)],
                             idsb[si], sem_i[si])

        def ids_wait(c, si):
            pltpu.make_async_copy(ids_hbm.at[pl.ds(r0 + c * CHUNK, CHUNK)],
                                  idsb[si], sem_i[si]).wait()

        def gather(si):
            pltpu.async_copy(item_hbm.at[idsb[si]], buf[si], sem_g[si])

        def gather_wait(si):
            pltpu.make_async_copy(item_hbm.at[idsb[si]], buf[si],
                                  sem_g[si]).wait()

        def wb(c, si):
            pltpu.async_copy(buf[si],
                             out_hbm.at[pl.ds(r0 + c * CHUNK, CHUNK)],
                             sem_w[si])

        def wb_wait(c, si):
            pltpu.make_async_copy(buf[si],
                                  out_hbm.at[pl.ds(r0 + c * CHUNK, CHUNK)],
                                  sem_w[si]).wait()

        for c in range(NB):
            ids_dma(c, c)
        for c in range(2):
            ids_wait(c, c)
            gather(c)

        def group_body(g, carry):
            for b in range(NB):
                c = g * NB + b
                gather_wait(b)
                @pl.when(c + NB < t_chunks)
                def _():
                    ids_dma(c + NB, b)

                cw0 = c * CHUNK
                bl0 = cw0 // SEQ
                bl1 = jnp.minimum(bl0 + 1, b_per_w - 1)
                n1 = jnp.minimum((bl0 + 1) * SEQ - cw0, CHUNK)
                soff0 = cw0 - bl0 * SEQ
                soff1 = cw0 - (bl0 + 1) * SEQ
                u0 = [user_v[bl0, pl.ds(LANES * j, LANES)] for j in range(NVEC)]
                u1 = [user_v[bl1, pl.ds(LANES * j, LANES)] for j in range(NVEC)]
                ib = buf[b]

                @plsc.parallel_loop(0, CHUNK, unroll=UNROLL)
                def _row(i):
                    first = i < n1
                    srow = jnp.where(first, soff0, soff1) + i
                    for j in range(NVEC):
                        sl = pl.ds(LANES * j, LANES)
                        uj = jnp.where(first, u0[j], u1[j])
                        ib[i, sl] = ib[i, sl] + pos_v[srow, sl] + uj

                wb(c, b)
                @pl.when(c + 2 < t_chunks)
                def _():
                    s2 = (b + 2) % NB
                    @pl.when(c >= 2)
                    def _():
                        wb_wait(c - 2, s2)
                    ids_wait(c + 2, s2)
                    gather(s2)
            return carry

        lax.fori_loop(0, groups, group_body, 0)
        wb_wait(t_chunks - 2, (t_chunks - 2) % NB)
        wb_wait(t_chunks - 1, (t_chunks - 1) % NB)

    return k


def kernel(input_ids, label_ids, item_table, label_table, pos_table):
    info = plsc.get_sparse_core_info()
    k = _make_kernel(info.num_cores, info.num_subcores)
    out = k(input_ids.astype(jnp.int32).reshape(-1),
            label_ids.astype(jnp.int32), item_table, label_table, pos_table)
    return out.reshape(BATCH, SEQ, EMBED)


# speedup vs baseline: 1.0287x; 1.0287x over previous
import functools

import jax
import jax.numpy as jnp
from jax import lax
from jax.experimental import pallas as pl
from jax.experimental.pallas import tpu as pltpu
from jax.experimental.pallas import tpu_sc as plsc

BATCH = 4096
SEQ = 200
EMBED = 128
LANES = 16
NVEC = EMBED // LANES

CHUNK = 128
NB = 4
UNROLL = 4
POS2 = 320


def _make_kernel(num_cores, num_subcores):
    nw = num_cores * num_subcores
    rows_w = BATCH * SEQ // nw
    b_per_w = BATCH // nw
    t_chunks = rows_w // CHUNK
    groups = t_chunks // NB

    mesh = plsc.VectorSubcoreMesh(core_axis_name="c", subcore_axis_name="s")

    @functools.partial(
        pl.kernel,
        mesh=mesh,
        out_type=jax.ShapeDtypeStruct((BATCH * SEQ, EMBED), jnp.float32),
        scratch_types=[
            pltpu.VMEM((b_per_w,), jnp.int32),
            pltpu.VMEM((b_per_w, EMBED), jnp.float32),
            pltpu.VMEM((SEQ, EMBED), jnp.float32),
        ]
        + [pltpu.VMEM((CHUNK,), jnp.int32) for _ in range(NB)]
        + [pltpu.VMEM((CHUNK, EMBED), jnp.float32) for _ in range(NB)]
        + [pltpu.SemaphoreType.DMA for _ in range(3 * NB + 1)],
    )
    def k(ids_hbm, labels_hbm, item_hbm, ltab_hbm, pos_hbm, out_hbm, *scr):
        labs_v, user_v, pos_v = scr[:3]
        idsb = scr[3:3 + NB]
        buf = scr[3 + NB:3 + 2 * NB]
        sem_i = scr[3 + 2 * NB:3 + 3 * NB]
        sem_g = scr[3 + 3 * NB:3 + 4 * NB]
        sem_w = scr[3 + 4 * NB:3 + 5 * NB]
        sem0 = scr[3 + 5 * NB]

        wid = lax.axis_index("s") * num_cores + lax.axis_index("c")
        r0 = wid * rows_w
        b0 = wid * b_per_w

        pltpu.sync_copy(labels_hbm.at[pl.ds(b0, b_per_w)], labs_v)
        pltpu.sync_copy(pos_hbm, pos_v)
        pltpu.async_copy(ltab_hbm.at[labs_v], user_v, sem0).wait()

        def ids_dma(c, si):
            pltpu.async_copy(ids_hbm.at[pl.ds(r0 + c * CHUNK, CHUNK)],
                             idsb[si], sem_i[si])

        def ids_wait(c, si):
            pltpu.make_async_copy(ids_hbm.at[pl.ds(r0 + c * CHUNK, CHUNK)],
                                  idsb[si], sem_i[si]).wait()

        def gather(si):
            pltpu.async_copy(item_hbm.at[idsb[si]], buf[si], sem_g[si])

        def gather_wait(si):
            pltpu.make_async_copy(item_hbm.at[idsb[si]], buf[si],
                                  sem_g[si]).wait()

        def wb(c, si):
            pltpu.async_copy(buf[si],
                             out_hbm.at[pl.ds(r0 + c * CHUNK, CHUNK)],
                             sem_w[si])

        def wb_wait(c, si):
            pltpu.make_async_copy(buf[si],
                                  out_hbm.at[pl.ds(r0 + c * CHUNK, CHUNK)],
                                  sem_w[si]).wait()

        for c in range(NB):
            ids_dma(c, c)
        for c in range(2):
            ids_wait(c, c)
            gather(c)

        def group_body(g, carry):
            for b in range(NB):
                c = g * NB + b
                gather_wait(b)
                @pl.when(c + NB < t_chunks)
                def _():
                    ids_dma(c + NB, b)

                cw0 = c * CHUNK
                bl0 = cw0 // SEQ
                bl1 = jnp.minimum(bl0 + 1, b_per_w - 1)
                n1 = jnp.minimum((bl0 + 1) * SEQ - cw0, CHUNK)
                soff0 = cw0 - bl0 * SEQ
                u0 = [user_v[bl0, pl.ds(LANES * j, LANES)] for j in range(NVEC)]
                u1 = [user_v[bl1, pl.ds(LANES * j, LANES)] for j in range(NVEC)]
                ib = buf[b]

                @plsc.parallel_loop(0, CHUNK, unroll=UNROLL)
                def _row(i):
                    first = i < n1
                    srow = jnp.where(first, soff0, soff0 - SEQ) + i
                    for j in range(NVEC):
                        sl = pl.ds(LANES * j, LANES)
                        uj = jnp.where(first, u0[j], u1[j])
                        ib[i, sl] = ib[i, sl] + pos_v[srow, sl] + uj

                wb(c, b)
                @pl.when(c + 2 < t_chunks)
                def _():
                    s2 = (b + 2) % NB
                    @pl.when(c >= 2)
                    def _():
                        wb_wait(c - 2, s2)
                    ids_wait(c + 2, s2)
                    gather(s2)
            return carry

        lax.fori_loop(0, groups, group_body, 0)
        wb_wait(t_chunks - 2, (t_chunks - 2) % NB)
        wb_wait(t_chunks - 1, (t_chunks - 1) % NB)

    return k


def kernel(input_ids, label_ids, item_table, label_table, pos_table):
    info = plsc.get_sparse_core_info()
    k = _make_kernel(info.num_cores, info.num_subcores)
    out = k(input_ids.astype(jnp.int32).reshape(-1),
            label_ids.astype(jnp.int32), item_table, label_table, pos_table)
    return out.reshape(BATCH, SEQ, EMBED)


# speedup vs baseline: 1.2057x; 1.1721x over previous
import functools

import jax
import jax.numpy as jnp
from jax import lax
from jax.experimental import pallas as pl
from jax.experimental.pallas import tpu as pltpu
from jax.experimental.pallas import tpu_sc as plsc

BATCH = 4096
SEQ = 200
EMBED = 128
LANES = 16
NVEC = EMBED // LANES

CHUNK = 128
NB = 5
SUPER = 25
UNROLL = 4


def _make_kernel(num_cores, num_subcores):
    nw = num_cores * num_subcores
    rows_w = BATCH * SEQ // nw
    b_per_w = BATCH // nw
    t_chunks = rows_w // CHUNK
    groups = t_chunks // SUPER

    mesh = plsc.VectorSubcoreMesh(core_axis_name="c", subcore_axis_name="s")

    @functools.partial(
        pl.kernel,
        mesh=mesh,
        out_type=jax.ShapeDtypeStruct((BATCH * SEQ, EMBED), jnp.float32),
        scratch_types=[
            pltpu.VMEM((b_per_w,), jnp.int32),
            pltpu.VMEM((b_per_w, EMBED), jnp.float32),
            pltpu.VMEM((SEQ, EMBED), jnp.float32),
        ]
        + [pltpu.VMEM((CHUNK,), jnp.int32) for _ in range(NB)]
        + [pltpu.VMEM((CHUNK, EMBED), jnp.float32) for _ in range(NB)]
        + [pltpu.SemaphoreType.DMA for _ in range(3 * NB + 1)],
    )
    def k(ids_hbm, labels_hbm, item_hbm, ltab_hbm, pos_hbm, out_hbm, *scr):
        labs_v, user_v, pos_v = scr[:3]
        idsb = scr[3:3 + NB]
        buf = scr[3 + NB:3 + 2 * NB]
        sem_i = scr[3 + 2 * NB:3 + 3 * NB]
        sem_g = scr[3 + 3 * NB:3 + 4 * NB]
        sem_w = scr[3 + 4 * NB:3 + 5 * NB]
        sem0 = scr[3 + 5 * NB]

        wid = lax.axis_index("s") * num_cores + lax.axis_index("c")
        r0 = wid * rows_w
        b0 = wid * b_per_w

        pltpu.sync_copy(labels_hbm.at[pl.ds(b0, b_per_w)], labs_v)
        pltpu.sync_copy(pos_hbm, pos_v)
        pltpu.async_copy(ltab_hbm.at[labs_v], user_v, sem0).wait()

        def ids_dma(c, si):
            pltpu.async_copy(ids_hbm.at[pl.ds(r0 + c * CHUNK, CHUNK)],
                             idsb[si], sem_i[si])

        def ids_wait(c, si):
            pltpu.make_async_copy(ids_hbm.at[pl.ds(r0 + c * CHUNK, CHUNK)],
                                  idsb[si], sem_i[si]).wait()

        def gather(si):
            pltpu.async_copy(item_hbm.at[idsb[si]], buf[si], sem_g[si])

        def gather_wait(si):
            pltpu.make_async_copy(item_hbm.at[idsb[si]], buf[si],
                                  sem_g[si]).wait()

        def wb(c, si):
            pltpu.async_copy(buf[si],
                             out_hbm.at[pl.ds(r0 + c * CHUNK, CHUNK)],
                             sem_w[si])

        def wb_wait(c, si):
            pltpu.make_async_copy(buf[si],
                                  out_hbm.at[pl.ds(r0 + c * CHUNK, CHUNK)],
                                  sem_w[si]).wait()

        for c in range(NB):
            ids_dma(c, c)
        for c in range(2):
            ids_wait(c, c)
            gather(c)

        def group_body(g, carry):
            for p in range(SUPER):
                c = g * SUPER + p
                sl5 = p % NB
                soff0 = (CHUNK * p) % SEQ
                n1 = min(SEQ - soff0, CHUNK)
                q = (CHUNK * p) // SEQ
                bl0 = g * (CHUNK * SUPER // SEQ) + q

                gather_wait(sl5)
                @pl.when(c + NB < t_chunks)
                def _():
                    ids_dma(c + NB, sl5)

                u0 = [user_v[bl0, pl.ds(LANES * j, LANES)]
                      for j in range(NVEC)]
                ib = buf[sl5]

                @plsc.parallel_loop(0, n1, unroll=UNROLL)
                def _row0(i):
                    for j in range(NVEC):
                        sl = pl.ds(LANES * j, LANES)
                        ib[i, sl] = ib[i, sl] + pos_v[soff0 + i, sl] + u0[j]

                if n1 < CHUNK:
                    u1 = [user_v[bl0 + 1, pl.ds(LANES * j, LANES)]
                          for j in range(NVEC)]

                    @plsc.parallel_loop(n1, CHUNK, unroll=2)
                    def _row1(i):
                        for j in range(NVEC):
                            sl = pl.ds(LANES * j, LANES)
                            ib[i, sl] = ib[i, sl] + pos_v[i - n1, sl] + u1[j]

                wb(c, sl5)
                @pl.when(c + 2 < t_chunks)
                def _():
                    s2 = (p + 2) % NB
                    @pl.when(c >= NB - 2)
                    def _():
                        wb_wait(c - (NB - 2), s2)
                    ids_wait(c + 2, s2)
                    gather(s2)
            return carry

        lax.fori_loop(0, groups, group_body, 0)
        for c in range(t_chunks - (NB - 2), t_chunks):
            wb_wait(c, c % NB)

    return k


def kernel(input_ids, label_ids, item_table, label_table, pos_table):
    info = plsc.get_sparse_core_info()
    k = _make_kernel(info.num_cores, info.num_subcores)
    out = k(input_ids.astype(jnp.int32).reshape(-1),
            label_ids.astype(jnp.int32), item_table, label_table, pos_table)
    return out.reshape(BATCH, SEQ, EMBED)


# speedup vs baseline: 1.2703x; 1.0536x over previous
import functools

import jax
import jax.numpy as jnp
from jax import lax
from jax.experimental import pallas as pl
from jax.experimental.pallas import tpu as pltpu
from jax.experimental.pallas import tpu_sc as plsc

BATCH = 4096
SEQ = 200
EMBED = 128
LANES = 16
NVEC = EMBED // LANES

CHUNK = 128
NB = 5
SUPER = 25
UNROLL = 4


def _make_kernel(num_cores, num_subcores):
    nw = num_cores * num_subcores
    rows_w = BATCH * SEQ // nw
    b_per_w = BATCH // nw
    t_chunks = rows_w // CHUNK
    groups = t_chunks // SUPER

    mesh = plsc.VectorSubcoreMesh(core_axis_name="c", subcore_axis_name="s")

    @functools.partial(
        pl.kernel,
        mesh=mesh,
        out_type=jax.ShapeDtypeStruct((BATCH * SEQ, EMBED), jnp.float32),
        scratch_types=[
            pltpu.VMEM((b_per_w,), jnp.int32),
            pltpu.VMEM((b_per_w, EMBED), jnp.float32),
            pltpu.VMEM((SEQ, EMBED // 2), jnp.int32),
        ]
        + [pltpu.VMEM((CHUNK,), jnp.int32) for _ in range(NB)]
        + [pltpu.VMEM((CHUNK, EMBED), jnp.float32) for _ in range(NB)]
        + [pltpu.SemaphoreType.DMA for _ in range(3 * NB + 1)],
    )
    def k(ids_hbm, labels_hbm, item_hbm, ltab_hbm, pos_hbm, out_hbm, *scr):
        labs_v, user_v, pos_v = scr[:3]
        idsb = scr[3:3 + NB]
        buf = scr[3 + NB:3 + 2 * NB]
        sem_i = scr[3 + 2 * NB:3 + 3 * NB]
        sem_g = scr[3 + 3 * NB:3 + 4 * NB]
        sem_w = scr[3 + 4 * NB:3 + 5 * NB]
        sem0 = scr[3 + 5 * NB]

        wid = lax.axis_index("s") * num_cores + lax.axis_index("c")
        r0 = wid * rows_w
        b0 = wid * b_per_w

        pltpu.sync_copy(labels_hbm.at[pl.ds(b0, b_per_w)], labs_v)
        pltpu.sync_copy(pos_hbm, pos_v)
        pltpu.async_copy(ltab_hbm.at[labs_v], user_v, sem0).wait()

        def ids_dma(c, si):
            pltpu.async_copy(ids_hbm.at[pl.ds(r0 + c * CHUNK, CHUNK)],
                             idsb[si], sem_i[si])

        def ids_wait(c, si):
            pltpu.make_async_copy(ids_hbm.at[pl.ds(r0 + c * CHUNK, CHUNK)],
                                  idsb[si], sem_i[si]).wait()

        def gather(si):
            pltpu.async_copy(item_hbm.at[idsb[si]], buf[si], sem_g[si])

        def gather_wait(si):
            pltpu.make_async_copy(item_hbm.at[idsb[si]], buf[si],
                                  sem_g[si]).wait()

        def wb(c, si):
            pltpu.async_copy(buf[si],
                             out_hbm.at[pl.ds(r0 + c * CHUNK, CHUNK)],
                             sem_w[si])

        def wb_wait(c, si):
            pltpu.make_async_copy(buf[si],
                                  out_hbm.at[pl.ds(r0 + c * CHUNK, CHUNK)],
                                  sem_w[si]).wait()

        for c in range(NB):
            ids_dma(c, c)
        for c in range(2):
            ids_wait(c, c)
            gather(c)

        def group_body(g, carry):
            for p in range(SUPER):
                c = g * SUPER + p
                sl5 = p % NB
                soff0 = (CHUNK * p) % SEQ
                n1 = min(SEQ - soff0, CHUNK)
                q = (CHUNK * p) // SEQ
                bl0 = g * (CHUNK * SUPER // SEQ) + q

                gather_wait(sl5)
                @pl.when(c + NB < t_chunks)
                def _():
                    ids_dma(c + NB, sl5)

                u0 = [user_v[bl0, pl.ds(LANES * j, LANES)]
                      for j in range(NVEC)]
                ib = buf[sl5]

                @plsc.parallel_loop(0, n1, unroll=UNROLL)
                def _row0(i):
                    for m in range(NVEC // 2):
                        xb = pos_v[soff0 + i, pl.ds(LANES * m, LANES)]
                        a = lax.bitcast_convert_type(xb << 16, jnp.float32)
                        bb = lax.bitcast_convert_type(
                            xb & jnp.int32(-65536), jnp.float32)
                        s0 = pl.ds(2 * LANES * m, LANES)
                        s1 = pl.ds(2 * LANES * m + LANES, LANES)
                        ib[i, s0] = ib[i, s0] + a + u0[2 * m]
                        ib[i, s1] = ib[i, s1] + bb + u0[2 * m + 1]

                if n1 < CHUNK:
                    u1 = [user_v[bl0 + 1, pl.ds(LANES * j, LANES)]
                          for j in range(NVEC)]

                    @plsc.parallel_loop(n1, CHUNK, unroll=2)
                    def _row1(i):
                        for m in range(NVEC // 2):
                            xb = pos_v[i - n1, pl.ds(LANES * m, LANES)]
                            a = lax.bitcast_convert_type(xb << 16, jnp.float32)
                            bb = lax.bitcast_convert_type(
                                xb & jnp.int32(-65536), jnp.float32)
                            s0 = pl.ds(2 * LANES * m, LANES)
                            s1 = pl.ds(2 * LANES * m + LANES, LANES)
                            ib[i, s0] = ib[i, s0] + a + u1[2 * m]
                            ib[i, s1] = ib[i, s1] + bb + u1[2 * m + 1]

                wb(c, sl5)
                @pl.when(c + 2 < t_chunks)
                def _():
                    s2 = (p + 2) % NB
                    @pl.when(c >= NB - 2)
                    def _():
                        wb_wait(c - (NB - 2), s2)
                    ids_wait(c + 2, s2)
                    gather(s2)
            return carry

        lax.fori_loop(0, groups, group_body, 0)
        for c in range(t_chunks - (NB - 2), t_chunks):
            wb_wait(c, c % NB)

    return k


def kernel(input_ids, label_ids, item_table, label_table, pos_table):
    info = plsc.get_sparse_core_info()
    k = _make_kernel(info.num_cores, info.num_subcores)
    pu16 = jax.lax.bitcast_convert_type(
        pos_table.astype(jnp.bfloat16), jnp.uint16)
    r = pu16.reshape(SEQ, NVEC // 2, 2, LANES).astype(jnp.uint32)
    pos_pack = jax.lax.bitcast_convert_type(
        r[:, :, 0, :] | (r[:, :, 1, :] << 16), jnp.int32
    ).reshape(SEQ, EMBED // 2)
    out = k(input_ids.astype(jnp.int32).reshape(-1),
            label_ids.astype(jnp.int32), item_table, label_table, pos_pack)
    return out.reshape(BATCH, SEQ, EMBED)
